# Initial kernel scaffold; baseline (speedup 1.0000x reference)
import jax
import jax.numpy as jnp
from jax.experimental import pallas as pl


def kernel(x, edge_index, W, att_src, att_dst, bias):
    raise NotImplementedError("write your pallas kernel here")


# speedup vs baseline: 39.9638x; 39.9638x over previous
import jax
import jax.numpy as jnp
from jax import lax
from jax.experimental import pallas as pl
from jax.experimental.pallas import tpu as pltpu
from jax.experimental.pallas import tpu_sc as plsc

N = 10000
D = 128
H = 8
C = 16
NC = 2
NS = 16
NW = NC * NS
B = 128

NPAD = 10240
RPW = NPAD // NW
RPS = NPAD // NS


def _bcast_lane(row16, lane):
    idx = jnp.full((16, 1), lane, jnp.int32)
    dnums = lax.GatherDimensionNumbers(
        offset_dims=(), collapsed_slice_dims=(0,), start_index_map=(0,))
    return lax.gather(row16, idx, dnums, (1,),
                      mode=lax.GatherScatterMode.PROMISE_IN_BOUNDS)


def _tc_proj_body(x_ref, wt_ref, ss_ref, sd_ref,
                  h_ref, as_ref, ad_ref, mxs_ref, mxd_ref):
    i = pl.program_id(0)
    h = jnp.dot(x_ref[...], wt_ref[...], preferred_element_type=jnp.float32)
    h_ref[...] = h
    a_s = jnp.dot(h, ss_ref[...], preferred_element_type=jnp.float32)
    a_d = jnp.dot(h, sd_ref[...], preferred_element_type=jnp.float32)
    as_ref[...] = a_s
    ad_ref[...] = a_d
    ms = jnp.broadcast_to(jnp.max(a_s, axis=0, keepdims=True), (8, 2 * H))
    md = jnp.broadcast_to(jnp.max(a_d, axis=0, keepdims=True), (8, 2 * H))

    @pl.when(i == 0)
    def _():
        mxs_ref[...] = ms
        mxd_ref[...] = md

    @pl.when(i > 0)
    def _():
        mxs_ref[...] = jnp.maximum(mxs_ref[...], ms)
        mxd_ref[...] = jnp.maximum(mxd_ref[...], md)


def _tc_proj(xp, wt, s_src, s_dst):
    rb = 256
    grid = (NPAD // rb,)
    return pl.pallas_call(
        _tc_proj_body,
        grid=grid,
        in_specs=[
            pl.BlockSpec((rb, D), lambda i: (i, 0)),
            pl.BlockSpec((D, D), lambda i: (0, 0)),
            pl.BlockSpec((D, 2 * H), lambda i: (0, 0)),
            pl.BlockSpec((D, 2 * H), lambda i: (0, 0)),
        ],
        out_specs=[
            pl.BlockSpec((rb, D), lambda i: (i, 0)),
            pl.BlockSpec((rb, 2 * H), lambda i: (i, 0)),
            pl.BlockSpec((rb, 2 * H), lambda i: (i, 0)),
            pl.BlockSpec((8, 2 * H), lambda i: (0, 0)),
            pl.BlockSpec((8, 2 * H), lambda i: (0, 0)),
        ],
        out_shape=[
            jax.ShapeDtypeStruct((NPAD, D), jnp.float32),
            jax.ShapeDtypeStruct((NPAD, 2 * H), jnp.float32),
            jax.ShapeDtypeStruct((NPAD, 2 * H), jnp.float32),
            jax.ShapeDtypeStruct((8, 2 * H), jnp.float32),
            jax.ShapeDtypeStruct((8, 2 * H), jnp.float32),
        ],
    )(xp, wt, s_src, s_dst)


def _edge_accum(src, dst, atab_s, atab_d, htab, g16, z128, z16, epad):
    epw = epad // NW
    nblk = epw // B
    mesh = plsc.VectorSubcoreMesh(core_axis_name="c", subcore_axis_name="s",
                                  num_cores=NC, num_subcores=NS)

    def body(src_h, dst_h, as_h, ad_h, h_h, g_h, z128_h, z16_h,
             exp_h, macc_h, sacc_h,
             msg_s, sum_s, srcv, dstv, asv, adv, hv, ev, gv,
             sem1, sem2, sem3):
        cid = lax.axis_index("c")
        sid = lax.axis_index("s")
        wid = cid * NS + sid
        rows0 = sid * RPS
        pltpu.sync_copy(z128_h.at[pl.ds(rows0, RPS)], msg_s.at[pl.ds(rows0, RPS)])
        pltpu.sync_copy(z16_h.at[pl.ds(rows0, RPS)], sum_s.at[pl.ds(rows0, RPS)])
        pltpu.sync_copy(g_h, gv)
        plsc.subcore_barrier()
        g = gv[...]

        def block(b, _):
            base = wid * epw + b * B
            pltpu.sync_copy(src_h.at[pl.ds(base, B)], srcv)
            pltpu.sync_copy(dst_h.at[pl.ds(base, B)], dstv)
            cp_a = pltpu.async_copy(as_h.at[srcv], asv, sem1)
            cp_d = pltpu.async_copy(ad_h.at[dstv], adv, sem2)
            cp_h = pltpu.async_copy(h_h.at[srcv], hv, sem3)
            cp_a.wait()
            cp_d.wait()

            def exp_i(j, _):
                t = asv[j, :] + adv[j, :]
                t = jnp.maximum(t, t * jnp.float32(0.2))
                ev[j, :] = jnp.exp(t - g)
                return 0

            lax.fori_loop(0, B, exp_i, 0, unroll=4)
            pltpu.sync_copy(ev, exp_h.at[pl.ds(base, B)])
            pltpu.sync_copy(ev, sum_s.at[dstv], add=True)
            cp_h.wait()

            def msg_i(e, _):
                erow = ev[e, :]
                for hh in range(H):
                    coef = _bcast_lane(erow, hh)
                    hv[e, pl.ds(hh * C, C)] = hv[e, pl.ds(hh * C, C)] * coef
                return 0

            lax.fori_loop(0, B, msg_i, 0)
            pltpu.sync_copy(hv, msg_s.at[dstv], add=True)
            return 0

        lax.fori_loop(0, nblk, block, 0)
        plsc.subcore_barrier()
        pltpu.sync_copy(msg_s.at[pl.ds(rows0, RPS)], macc_h.at[cid, pl.ds(rows0, RPS)])
        pltpu.sync_copy(sum_s.at[pl.ds(rows0, RPS)], sacc_h.at[cid, pl.ds(rows0, RPS)])

    f = pl.kernel(
        body,
        out_type=(
            jax.ShapeDtypeStruct((epad, 2 * H), jnp.float32),
            jax.ShapeDtypeStruct((NC, NPAD, D), jnp.float32),
            jax.ShapeDtypeStruct((NC, NPAD, 2 * H), jnp.float32),
        ),
        mesh=mesh,
        compiler_params=pltpu.CompilerParams(use_tc_tiling_on_sc=False),
        scratch_types=[
            pltpu.VMEM_SHARED((NPAD, D), jnp.float32),
            pltpu.VMEM_SHARED((NPAD, 2 * H), jnp.float32),
            pltpu.VMEM((B,), jnp.int32),
            pltpu.VMEM((B,), jnp.int32),
            pltpu.VMEM((B, 2 * H), jnp.float32),
            pltpu.VMEM((B, 2 * H), jnp.float32),
            pltpu.VMEM((B, D), jnp.float32),
            pltpu.VMEM((B, 2 * H), jnp.float32),
            pltpu.VMEM((16,), jnp.float32),
            pltpu.SemaphoreType.DMA,
            pltpu.SemaphoreType.DMA,
            pltpu.SemaphoreType.DMA,
        ],
    )
    return f(src, dst, atab_s, atab_d, htab, g16, z128, z16)


def _node_combine(macc, sacc, bias):
    mesh = plsc.VectorSubcoreMesh(core_axis_name="c", subcore_axis_name="s",
                                  num_cores=NC, num_subcores=NS)

    def body(macc_h, sacc_h, bias_h,
             out_h, inv_h,
             m0, m1, s0, s1, invv, biasv, sem1, sem2):
        cid = lax.axis_index("c")
        sid = lax.axis_index("s")
        wid = cid * NS + sid
        r0 = wid * RPW
        cp0 = pltpu.async_copy(macc_h.at[0, pl.ds(r0, RPW)], m0, sem1)
        cp1 = pltpu.async_copy(macc_h.at[1, pl.ds(r0, RPW)], m1, sem2)
        pltpu.sync_copy(sacc_h.at[0, pl.ds(r0, RPW)], s0)
        pltpu.sync_copy(sacc_h.at[1, pl.ds(r0, RPW)], s1)
        pltpu.sync_copy(bias_h, biasv)

        def inv_i(j, _):
            s = s0[j, :] + s1[j, :]
            invv[j, :] = jnp.float32(1.0) / (s + jnp.float32(1e-16))
            return 0

        lax.fori_loop(0, RPW, inv_i, 0, unroll=4)
        pltpu.sync_copy(invv, inv_h.at[pl.ds(r0, RPW)])
        cp0.wait()
        cp1.wait()

        def out_i(e, _):
            irow = invv[e, :]
            for hh in range(H):
                iv = _bcast_lane(irow, hh)
                m = m0[e, pl.ds(hh * C, C)] + m1[e, pl.ds(hh * C, C)]
                m0[e, pl.ds(hh * C, C)] = m * iv + biasv[pl.ds(hh * C, C)]
            return 0

        lax.fori_loop(0, RPW, out_i, 0)
        pltpu.sync_copy(m0, out_h.at[pl.ds(r0, RPW)])

    f = pl.kernel(
        body,
        out_type=(
            jax.ShapeDtypeStruct((NPAD, D), jnp.float32),
            jax.ShapeDtypeStruct((NPAD, 2 * H), jnp.float32),
        ),
        mesh=mesh,
        compiler_params=pltpu.CompilerParams(use_tc_tiling_on_sc=False),
        scratch_types=[
            pltpu.VMEM((RPW, D), jnp.float32),
            pltpu.VMEM((RPW, D), jnp.float32),
            pltpu.VMEM((RPW, 2 * H), jnp.float32),
            pltpu.VMEM((RPW, 2 * H), jnp.float32),
            pltpu.VMEM((RPW, 2 * H), jnp.float32),
            pltpu.VMEM((D,), jnp.float32),
            pltpu.SemaphoreType.DMA,
            pltpu.SemaphoreType.DMA,
        ],
    )
    return f(macc, sacc, bias)


def _alpha_norm(dst, exp16, inv, epad):
    epw = epad // NW
    nblk = epw // B
    mesh = plsc.VectorSubcoreMesh(core_axis_name="c", subcore_axis_name="s",
                                  num_cores=NC, num_subcores=NS)

    def body(dst_h, exp_h, inv_h, alpha_h,
             dstv, e16v, ivv, av, sem1):
        cid = lax.axis_index("c")
        sid = lax.axis_index("s")
        wid = cid * NS + sid

        def block(b, _):
            base = wid * epw + b * B
            pltpu.sync_copy(dst_h.at[pl.ds(base, B)], dstv)
            cp = pltpu.async_copy(inv_h.at[dstv], ivv, sem1)
            pltpu.sync_copy(exp_h.at[pl.ds(base, B)], e16v)
            cp.wait()

            def mul_i(j, _):
                av[j, :] = e16v[j, :] * ivv[j, :]
                return 0

            lax.fori_loop(0, B, mul_i, 0, unroll=4)
            pltpu.sync_copy(av, alpha_h.at[pl.ds(base, B)])
            return 0

        lax.fori_loop(0, nblk, block, 0)

    f = pl.kernel(
        body,
        out_type=jax.ShapeDtypeStruct((epad, 2 * H), jnp.float32),
        mesh=mesh,
        compiler_params=pltpu.CompilerParams(use_tc_tiling_on_sc=False),
        scratch_types=[
            pltpu.VMEM((B,), jnp.int32),
            pltpu.VMEM((B, 2 * H), jnp.float32),
            pltpu.VMEM((B, 2 * H), jnp.float32),
            pltpu.VMEM((B, 2 * H), jnp.float32),
            pltpu.SemaphoreType.DMA,
        ],
    )
    return f(dst, exp16, inv)


def kernel(x, edge_index, W, att_src, att_dst, bias):
    n = x.shape[0]
    e = edge_index.shape[1]
    ne = e + n
    epad = ((ne + NW * B - 1) // (NW * B)) * (NW * B)

    loop = jnp.arange(n, dtype=edge_index.dtype)
    ei = jnp.concatenate([edge_index, jnp.stack([loop, loop], axis=0)], axis=1)
    padi = jnp.full((epad - ne,), n, jnp.int32)
    src = jnp.concatenate([ei[0], padi])
    dst = jnp.concatenate([ei[1], padi])

    xp = jnp.pad(x, ((0, NPAD - n), (0, 0)))
    hsel = jnp.repeat(jnp.arange(H), C)
    eye = jax.nn.one_hot(hsel, H, dtype=jnp.float32)
    s_src1 = eye * att_src.reshape(-1)[:, None]
    s_dst1 = eye * att_dst.reshape(-1)[:, None]
    s_src = jnp.concatenate([s_src1, s_src1], axis=1)
    s_dst = jnp.concatenate([s_dst1, s_dst1], axis=1)

    htab, atab_s, atab_d, mxs, mxd = _tc_proj(xp, W.T, s_src, s_dst)

    t = mxs[0] + mxd[0]
    g16 = jnp.where(t > 0, t, 0.2 * t)

    z128 = jnp.zeros((NPAD, D), jnp.float32)
    z16 = jnp.zeros((NPAD, 2 * H), jnp.float32)

    exp16, macc, sacc = _edge_accum(src, dst, atab_s, atab_d, htab,
                                    g16, z128, z16, epad)
    out_full, inv = _node_combine(macc, sacc, bias)
    alpha16 = _alpha_norm(dst, exp16, inv, epad)

    out = out_full[:n]
    alpha = alpha16[:ne, :H]
    return out, ei, alpha
